# baseline (speedup 1.0000x reference)
import functools

import jax
import jax.numpy as jnp
from jax import lax
from jax.experimental import pallas as pl
from jax.experimental.pallas import tpu as pltpu
from jax.experimental.pallas import tpu_sc as plsc

_B = 16384
_D = 64
_LANES = 16


def _gmf_kernel(b_per_w, user_idx, item_idx, user_table,
                item_table, w_vec, bias, out_hbm,
                idx_u_v, idx_i_v, rows_u, rows_i, w_v, b_v, out_v, sem):
    n_chunks = b_per_w // 128
    wid = lax.axis_index("s") * 2 + lax.axis_index("c")
    base = wid * b_per_w

    for j in range(n_chunks):
        pltpu.sync_copy(user_idx.at[pl.ds(base + j * 128, 128)], idx_u_v.at[j])
        pltpu.sync_copy(item_idx.at[pl.ds(base + j * 128, 128)], idx_i_v.at[j])
    pltpu.sync_copy(w_vec, w_v)
    pltpu.sync_copy(bias, b_v)

    copies = []
    for j in range(n_chunks):
        copies.append(pltpu.async_copy(
            user_table.at[idx_u_v.at[j]], rows_u.at[pl.ds(j * 128, 128)], sem))
        copies.append(pltpu.async_copy(
            item_table.at[idx_i_v.at[j]], rows_i.at[pl.ds(j * 128, 128)], sem))
    for c in copies:
        c.wait()

    w_chunks = [w_v[pl.ds(k * _LANES, _LANES)] for k in range(_D // _LANES)]
    lane = lax.iota(jnp.int32, _LANES)
    bias_vec = b_v[...]

    def group_body(g, carry):
        res = jnp.zeros((_LANES,), jnp.float32)
        for e in range(_LANES):
            idx = g * _LANES + e
            acc = (rows_u[idx, pl.ds(0, _LANES)]
                   * rows_i[idx, pl.ds(0, _LANES)] * w_chunks[0])
            for k in range(1, _D // _LANES):
                acc = acc + (rows_u[idx, pl.ds(k * _LANES, _LANES)]
                             * rows_i[idx, pl.ds(k * _LANES, _LANES)]
                             * w_chunks[k])
            res = jnp.where(lane == e, jnp.sum(acc), res)
        x = res + bias_vec
        out_v[pl.ds(g * _LANES, _LANES)] = 1.0 / (1.0 + jnp.exp(-x))
        return carry

    lax.fori_loop(0, b_per_w // _LANES, group_body, 0)

    pltpu.sync_copy(out_v, out_hbm.at[pl.ds(base, b_per_w)])


def kernel(user_input, item_input, user_table, item_table, W, b):
    info = plsc.get_sparse_core_info()
    num_workers = info.num_cores * info.num_subcores
    b_per_w = _B // num_workers
    n_chunks = b_per_w // 128

    mesh = plsc.VectorSubcoreMesh(core_axis_name="c", subcore_axis_name="s")
    run = pl.kernel(
        functools.partial(_gmf_kernel, b_per_w),
        mesh=mesh,
        compiler_params=pltpu.CompilerParams(
            needs_layout_passes=False, use_tc_tiling_on_sc=False),
        out_type=jax.ShapeDtypeStruct((_B,), jnp.float32),
        scratch_types=[
            pltpu.VMEM((n_chunks, 128), jnp.int32),
            pltpu.VMEM((n_chunks, 128), jnp.int32),
            pltpu.VMEM((b_per_w, _D), jnp.float32),
            pltpu.VMEM((b_per_w, _D), jnp.float32),
            pltpu.VMEM((_D,), jnp.float32),
            pltpu.VMEM((_LANES,), jnp.float32),
            pltpu.VMEM((b_per_w,), jnp.float32),
            pltpu.SemaphoreType.DMA,
        ],
    )
    out = run(user_input.astype(jnp.int32), item_input.astype(jnp.int32),
              user_table, item_table, W.reshape(_D),
              jnp.broadcast_to(b.reshape(1), (_LANES,)))
    return out.reshape(_B, 1)


# speedup vs baseline: 2.1878x; 2.1878x over previous
import functools

import jax
import jax.numpy as jnp
from jax import lax
from jax.experimental import pallas as pl
from jax.experimental.pallas import tpu as pltpu
from jax.experimental.pallas import tpu_sc as plsc

_B = 16384
_D = 64
_LANES = 16


def _gmf_kernel(b_per_w, user_idx, item_idx, user_table,
                item_table, w_vec, bias, out_hbm,
                idx_u_v, idx_i_v, buf_u, buf_i,
                w_v, b_v, out_v, sem0, sem1):
    n_ichunks = b_per_w // 128
    n_groups = b_per_w // _LANES
    wid = lax.axis_index("s") * 2 + lax.axis_index("c")
    base = wid * b_per_w

    for j in range(n_ichunks):
        pltpu.sync_copy(user_idx.at[pl.ds(base + j * 128, 128)], idx_u_v.at[j])
        pltpu.sync_copy(item_idx.at[pl.ds(base + j * 128, 128)], idx_i_v.at[j])
    pltpu.sync_copy(w_vec, w_v)
    pltpu.sync_copy(bias, b_v)

    sems = [sem0, sem1]

    def issue_group(g, slot):
        pos = g * _LANES
        iu = idx_u_v[pos // 128, pl.ds(pos % 128, _LANES)]
        ii = idx_i_v[pos // 128, pl.ds(pos % 128, _LANES)]
        bu = lax.shift_right_logical(iu, 3)
        bi = lax.shift_right_logical(ii, 3)
        for e in range(_LANES):
            pltpu.async_copy(user_table.at[bu[e]], buf_u.at[slot, e],
                             sems[slot])
            pltpu.async_copy(item_table.at[bi[e]], buf_i.at[slot, e],
                             sems[slot])

    def drain_group(slot):
        pltpu.make_async_copy(user_table.at[pl.ds(0, _LANES)],
                              buf_u.at[slot], sems[slot]).wait()
        pltpu.make_async_copy(item_table.at[pl.ds(0, _LANES)],
                              buf_i.at[slot], sems[slot]).wait()

    w_chunks = [w_v[pl.ds(k * _LANES, _LANES)] for k in range(_D // _LANES)]
    lane = lax.iota(jnp.int32, _LANES)
    bias_vec = b_v[...]

    def compute_group(g, slot):
        pos = g * _LANES
        ru = idx_u_v[pos // 128, pl.ds(pos % 128, _LANES)] & 7
        ri = idx_i_v[pos // 128, pl.ds(pos % 128, _LANES)] & 7
        res = jnp.zeros((_LANES,), jnp.float32)
        for e in range(_LANES):
            acc = (buf_u[slot, e, ru[e], pl.ds(0, _LANES)]
                   * buf_i[slot, e, ri[e], pl.ds(0, _LANES)] * w_chunks[0])
            for k in range(1, _D // _LANES):
                acc = acc + (buf_u[slot, e, ru[e], pl.ds(k * _LANES, _LANES)]
                             * buf_i[slot, e, ri[e],
                                     pl.ds(k * _LANES, _LANES)]
                             * w_chunks[k])
            res = jnp.where(lane == e, jnp.sum(acc), res)
        x = res + bias_vec
        out_v[pl.ds(pos, _LANES)] = 1.0 / (1.0 + jnp.exp(-x))

    issue_group(0, 0)

    def pipe_body(h, carry):
        g = h * 2
        issue_group(g + 1, 1)
        drain_group(0)
        compute_group(g, 0)

        @pl.when(g + 2 < n_groups)
        def _():
            issue_group(g + 2, 0)

        drain_group(1)
        compute_group(g + 1, 1)
        return carry

    lax.fori_loop(0, n_groups // 2, pipe_body, 0)

    pltpu.sync_copy(out_v, out_hbm.at[pl.ds(base, b_per_w)])


def kernel(user_input, item_input, user_table, item_table, W, b):
    info = plsc.get_sparse_core_info()
    num_workers = info.num_cores * info.num_subcores
    b_per_w = _B // num_workers
    n_ichunks = b_per_w // 128

    mesh = plsc.VectorSubcoreMesh(core_axis_name="c", subcore_axis_name="s")
    run = pl.kernel(
        functools.partial(_gmf_kernel, b_per_w),
        mesh=mesh,
        compiler_params=pltpu.CompilerParams(needs_layout_passes=False),
        out_type=jax.ShapeDtypeStruct((_B,), jnp.float32),
        scratch_types=[
            pltpu.VMEM((n_ichunks, 128), jnp.int32),
            pltpu.VMEM((n_ichunks, 128), jnp.int32),
            pltpu.VMEM((2, _LANES, 8, _D), jnp.float32),
            pltpu.VMEM((2, _LANES, 8, _D), jnp.float32),
            pltpu.VMEM((_D,), jnp.float32),
            pltpu.VMEM((_LANES,), jnp.float32),
            pltpu.VMEM((b_per_w,), jnp.float32),
            pltpu.SemaphoreType.DMA,
            pltpu.SemaphoreType.DMA,
        ],
    )
    out = run(user_input.astype(jnp.int32), item_input.astype(jnp.int32),
              user_table.reshape(-1, 8, _D), item_table.reshape(-1, 8, _D),
              W.reshape(_D), jnp.broadcast_to(b.reshape(1), (_LANES,)))
    return out.reshape(_B, 1)
